# Initial kernel scaffold; baseline (speedup 1.0000x reference)
import jax
import jax.numpy as jnp
from jax.experimental import pallas as pl


def kernel(s_sem, s_struct, rel_ids, dir_ids, topm, rel_emb, dir_emb, W1, b1, W2, b2):
    raise NotImplementedError("write your pallas kernel here")


# speedup vs baseline: 1.2774x; 1.2774x over previous
import functools

import jax
import jax.numpy as jnp
from jax import lax
from jax.experimental import pallas as pl
from jax.experimental.pallas import tpu as pltpu
from jax.experimental.pallas import tpu_sc as plsc

B = 128
N = 32768
L = 16
NCH = N // L
NC, NS = 2, 16
NW = NC * NS
RPW = B // NW
TOPK = 10


def _insert(xv, xi, tv, ti):
    ntv, nti = [], []
    for k in range(TOPK):
        c = xv > tv[k]
        ntv.append(jnp.where(c, xv, tv[k]))
        nti.append(jnp.where(c, xi, ti[k]))
        xv, xi = jnp.where(c, tv[k], xv), jnp.where(c, ti[k], xi)
    return ntv, nti


GRP = 8


def _scan_row(buf, lane_f, with_stats):
    zero = jnp.zeros((L,), jnp.float32)
    ninf = jnp.full((L,), -jnp.inf, jnp.float32)
    init = (zero, zero, ninf) + (ninf,) * TOPK + (zero,) * TOPK

    def body(g, carry):
        s, q, m = carry[0], carry[1], carry[2]
        tv = list(carry[3:3 + TOPK])
        ti = list(carry[3 + TOPK:])
        base = g * (L * GRP)
        xs = [buf[pl.ds(base + j * L, L)] for j in range(GRP)]
        gmax = xs[0]
        for j in range(1, GRP):
            gmax = jnp.maximum(gmax, xs[j])
        if with_stats:
            for x in xs:
                s = s + x
                q = q + x * x
            m = jnp.maximum(m, gmax)
        gf = g.astype(jnp.float32) * float(L * GRP)
        anyc = jnp.max(gmax - tv[TOPK - 1]) > 0.0

        def do(slots):
            tv_, ti_ = list(slots[:TOPK]), list(slots[TOPK:])
            for j in range(GRP):
                xi = gf + float(j * L) + lane_f
                tv_, ti_ = _insert(xs[j], xi, tv_, ti_)
            return tuple(tv_) + tuple(ti_)

        res = lax.cond(anyc, do, lambda slots: slots, tuple(tv) + tuple(ti))
        return (s, q, m) + res

    out = lax.fori_loop(0, NCH // GRP, body, init)
    return out[0], out[1], out[2], list(out[3:3 + TOPK]), list(out[3 + TOPK:])


def _softmax_partials(buf, m):
    zero = jnp.zeros((L,), jnp.float32)

    def body(i, carry):
        s1, s2 = carry
        x = buf[pl.ds(i * L, L)]
        e = jnp.exp(x - m)
        return (s1 + e, s2 + e * x)

    return lax.fori_loop(0, NCH, body, (zero, zero))


def _sc_body(sem_hbm, struct_hbm,
             part_hbm, sval_hbm, sidx_hbm, tval_hbm, tidx_hbm,
             sem_v, struct_v, part_v, sval_v, sidx_v, tval_v, tidx_v):
    wid = lax.axis_index("s") * NC + lax.axis_index("c")
    lane_f = lax.broadcasted_iota(jnp.int32, (L,), 0).astype(jnp.float32)

    for rr in range(RPW):
        r = wid * RPW + rr
        pltpu.sync_copy(sem_hbm.at[r], sem_v)
        pltpu.sync_copy(struct_hbm.at[r], struct_v)

        s, q, m, stv, sti = _scan_row(sem_v, lane_f, with_stats=True)
        s1, s2 = _softmax_partials(sem_v, m)
        _, _, _, ttv, tti = _scan_row(struct_v, lane_f, with_stats=False)

        part_v[pl.ds(0, L)] = s
        part_v[pl.ds(L, L)] = q
        part_v[pl.ds(2 * L, L)] = m
        part_v[pl.ds(3 * L, L)] = s1
        part_v[pl.ds(4 * L, L)] = s2
        for k in range(TOPK):
            sval_v[pl.ds(k * L, L)] = stv[k]
            sidx_v[pl.ds(k * L, L)] = sti[k]
            tval_v[pl.ds(k * L, L)] = ttv[k]
            tidx_v[pl.ds(k * L, L)] = tti[k]

        pltpu.sync_copy(part_v, part_hbm.at[r])
        pltpu.sync_copy(sval_v, sval_hbm.at[r])
        pltpu.sync_copy(sidx_v, sidx_hbm.at[r])
        pltpu.sync_copy(tval_v, tval_hbm.at[r])
        pltpu.sync_copy(tidx_v, tidx_hbm.at[r])


def _sc_stage(s_sem, s_struct):
    mesh = plsc.VectorSubcoreMesh(core_axis_name="c", subcore_axis_name="s",
                                  num_cores=NC, num_subcores=NS)
    f32 = jnp.float32
    out_type = [
        jax.ShapeDtypeStruct((B, 5 * L), f32),
        jax.ShapeDtypeStruct((B, TOPK * L), f32),
        jax.ShapeDtypeStruct((B, TOPK * L), f32),
        jax.ShapeDtypeStruct((B, TOPK * L), f32),
        jax.ShapeDtypeStruct((B, TOPK * L), f32),
    ]
    scratch = [
        pltpu.VMEM((N,), f32),
        pltpu.VMEM((N,), f32),
        pltpu.VMEM((5 * L,), f32),
        pltpu.VMEM((TOPK * L,), f32),
        pltpu.VMEM((TOPK * L,), f32),
        pltpu.VMEM((TOPK * L,), f32),
        pltpu.VMEM((TOPK * L,), f32),
    ]
    fn = pl.kernel(_sc_body, out_type=out_type, mesh=mesh,
                   compiler_params=pltpu.CompilerParams(
                       needs_layout_passes=False),
                   scratch_types=scratch)
    return fn(s_sem, s_struct)


def _tc_body(part_ref, sval_ref, sidx_ref, tval_ref, tidx_ref,
             relid_ref, dirid_ref, rel_emb_ref, dir_emb_ref,
             w1_ref, b1_ref, w2_ref, b2_ref, inv_ref, out_ref):
    nf = jnp.float32(N)
    part = part_ref[...]
    s = part[:, 0:L]
    q = part[:, L:2 * L]
    ml = part[:, 2 * L:3 * L]
    s1l = part[:, 3 * L:4 * L]
    s2l = part[:, 4 * L:5 * L]

    mean = jnp.sum(s, axis=1) / nf
    var = jnp.sum(q, axis=1) / nf - mean * mean
    std = jnp.sqrt(jnp.maximum(var, 0.0))
    maxv = jnp.max(ml, axis=1)
    w = jnp.exp(ml - maxv[:, None])
    s1 = jnp.sum(s1l * w, axis=1)
    s2 = jnp.sum(s2l * w, axis=1)
    ent = maxv + jnp.log(s1) - s2 / s1
    gap = maxv - mean

    def select10(vals, idx):
        v = vals
        sels = []
        for _ in range(TOPK):
            mx = jnp.max(v, axis=1, keepdims=True)
            ci = jnp.where(v == mx, idx, jnp.float32(1e9))
            si = jnp.min(ci, axis=1, keepdims=True)
            sels.append(si)
            v = jnp.where(idx == si, -jnp.inf, v)
        return sels

    semsel = select10(sval_ref[...], sidx_ref[...])
    strsel = jnp.concatenate(select10(tval_ref[...], tidx_ref[...]), axis=1)

    match = jnp.zeros((B,), jnp.float32)
    for i in range(TOPK):
        hit = jnp.max((semsel[i] == strsel).astype(jnp.float32), axis=1)
        match = match + hit
    agree = match * inv_ref[0, 0]

    rel_oh = (relid_ref[...] ==
              lax.broadcasted_iota(jnp.int32, (B, rel_emb_ref.shape[0]), 1)
              ).astype(jnp.float32)
    dir_oh = (dirid_ref[...] ==
              lax.broadcasted_iota(jnp.int32, (B, 2), 1)).astype(jnp.float32)
    rel_vec = jnp.dot(rel_oh, rel_emb_ref[...],
                      preferred_element_type=jnp.float32)
    dir_vec = jnp.dot(dir_oh, dir_emb_ref[...],
                      preferred_element_type=jnp.float32)

    x = jnp.concatenate(
        [mean[:, None], std[:, None], maxv[:, None], gap[:, None],
         ent[:, None], agree[:, None], rel_vec, dir_vec], axis=1)
    h = jnp.maximum(jnp.dot(x, w1_ref[...],
                            preferred_element_type=jnp.float32) + b1_ref[...],
                    0.0)
    z = jnp.dot(h, w2_ref[...], preferred_element_type=jnp.float32) + b2_ref[...]
    r = 1.0 / (1.0 + jnp.exp(-z))
    out_ref[...] = jnp.clip(r, 0.05, 0.95)


def kernel(s_sem, s_struct, rel_ids, dir_ids, topm, rel_emb, dir_emb,
           W1, b1, W2, b2):
    part, sval, sidx, tval, tidx = _sc_stage(s_sem, s_struct)

    inv_topm = (1.0 / jnp.asarray(topm, jnp.float32)).reshape(1, 1)
    relid = rel_ids.astype(jnp.int32).reshape(B, 1)
    dirid = dir_ids.astype(jnp.int32).reshape(B, 1)

    out = pl.pallas_call(
        _tc_body,
        out_shape=jax.ShapeDtypeStruct((B, 1), jnp.float32),
    )(part, sval, sidx, tval, tidx, relid, dirid, rel_emb, dir_emb,
      W1, b1.reshape(1, -1), W2, b2.reshape(1, 1), inv_topm)
    return out[:, 0]


# speedup vs baseline: 3.2991x; 2.5827x over previous
import functools

import jax
import jax.numpy as jnp
from jax import lax
from jax.experimental import pallas as pl
from jax.experimental.pallas import tpu as pltpu
from jax.experimental.pallas import tpu_sc as plsc

B = 128
N = 32768
L = 16
NCH = N // L
NC, NS = 2, 16
NW = NC * NS
RPW = B // NW
TOPK = 10


SCH = 16
SEG = NCH // SCH
XUNR = 8
SUNR = 4


def _pass1(buf, seg_val, seg_idx, lane_i, with_stats):
    zero = jnp.zeros((L,), jnp.float32)
    ninf = jnp.full((L,), -jnp.inf, jnp.float32)
    zi = jnp.zeros((L,), jnp.int32)

    def body(sg, carry):
        s, q, m = carry
        base = sg * (SCH * L)
        sm, smi = ninf, zi
        for j in range(SCH):
            x = buf[pl.ds(base + j * L, L)]
            gidx = base + (j * L) + lane_i
            c = x > sm
            sm = jnp.where(c, x, sm)
            smi = jnp.where(c, gidx, smi)
            if with_stats:
                s = s + x
                q = q + x * x
        m = jnp.maximum(m, sm)
        seg_val[pl.ds(sg * L, L)] = sm
        seg_idx[pl.ds(sg * L, L)] = smi
        return (s, q, m)

    return lax.fori_loop(0, SEG, body, (zero, zero, ninf))


def _softmax_partials(buf, m):
    zero = jnp.zeros((L,), jnp.float32)

    def body(g, carry):
        s1, s2 = carry
        base = g * (XUNR * L)
        for j in range(XUNR):
            x = buf[pl.ds(base + j * L, L)]
            e = jnp.exp(x - m)
            s1 = s1 + e
            s2 = s2 + e * x
        return (s1, s2)

    return lax.fori_loop(0, NCH // XUNR, body, (zero, zero))


def _extract10(buf, seg_val, seg_idx, lane_i):
    ninf = jnp.full((L,), -jnp.inf, jnp.float32)
    zi = jnp.zeros((L,), jnp.int32)
    outs_v, outs_i = [], []

    def scan(g, carry):
        bv, bi, bs = carry
        for u in range(SUNR):
            sg = g * SUNR + u
            v = seg_val[pl.ds(sg * L, L)]
            gi = seg_idx[pl.ds(sg * L, L)]
            c = v > bv
            bv = jnp.where(c, v, bv)
            bi = jnp.where(c, gi, bi)
            bs = jnp.where(c, zi + sg, bs)
        return (bv, bi, bs)

    for _ in range(TOPK):
        bv, bi, bs = lax.fori_loop(0, SEG // SUNR, scan, (ninf, zi, zi))
        outs_v.append(bv)
        outs_i.append(bi)
        plsc.store_scatter(buf, [bi], ninf)
        sbase = bs * (SCH * L) + lane_i
        nv, nvi = ninf, zi
        for j in range(SCH):
            gidx = sbase + j * L
            g = plsc.load_gather(buf, [gidx])
            c = g > nv
            nv = jnp.where(c, g, nv)
            nvi = jnp.where(c, gidx, nvi)
        sslot = bs * L + lane_i
        plsc.store_scatter(seg_val, [sslot], nv)
        plsc.store_scatter(seg_idx, [sslot], nvi)
    return outs_v, outs_i


def _sc_body(sem_hbm, struct_hbm,
             part_hbm, sval_hbm, sidx_hbm, tval_hbm, tidx_hbm,
             sem_v, struct_v, seg_val, seg_idx, part_v,
             sval_v, sidx_v, tval_v, tidx_v, dsem):
    wid = lax.axis_index("s") * NC + lax.axis_index("c")
    lane_i = lax.broadcasted_iota(jnp.int32, (L,), 0)

    for rr in range(RPW):
        r = wid * RPW + rr
        cp_struct = pltpu.async_copy(struct_hbm.at[r], struct_v, dsem)
        pltpu.sync_copy(sem_hbm.at[r], sem_v)

        s, q, m = _pass1(sem_v, seg_val, seg_idx, lane_i, with_stats=True)
        s1, s2 = _softmax_partials(sem_v, m)
        stv, sti = _extract10(sem_v, seg_val, seg_idx, lane_i)

        part_v[pl.ds(0, L)] = s
        part_v[pl.ds(L, L)] = q
        part_v[pl.ds(2 * L, L)] = m
        part_v[pl.ds(3 * L, L)] = s1
        part_v[pl.ds(4 * L, L)] = s2
        for k in range(TOPK):
            sval_v[pl.ds(k * L, L)] = stv[k]
            sidx_v[pl.ds(k * L, L)] = sti[k].astype(jnp.float32)

        cp_struct.wait()
        _pass1(struct_v, seg_val, seg_idx, lane_i, with_stats=False)
        ttv, tti = _extract10(struct_v, seg_val, seg_idx, lane_i)
        for k in range(TOPK):
            tval_v[pl.ds(k * L, L)] = ttv[k]
            tidx_v[pl.ds(k * L, L)] = tti[k].astype(jnp.float32)

        pltpu.sync_copy(part_v, part_hbm.at[r])
        pltpu.sync_copy(sval_v, sval_hbm.at[r])
        pltpu.sync_copy(sidx_v, sidx_hbm.at[r])
        pltpu.sync_copy(tval_v, tval_hbm.at[r])
        pltpu.sync_copy(tidx_v, tidx_hbm.at[r])


def _sc_stage(s_sem, s_struct):
    mesh = plsc.VectorSubcoreMesh(core_axis_name="c", subcore_axis_name="s",
                                  num_cores=NC, num_subcores=NS)
    f32 = jnp.float32
    out_type = [
        jax.ShapeDtypeStruct((B, 5 * L), f32),
        jax.ShapeDtypeStruct((B, TOPK * L), f32),
        jax.ShapeDtypeStruct((B, TOPK * L), f32),
        jax.ShapeDtypeStruct((B, TOPK * L), f32),
        jax.ShapeDtypeStruct((B, TOPK * L), f32),
    ]
    scratch = [
        pltpu.VMEM((N,), f32),
        pltpu.VMEM((N,), f32),
        pltpu.VMEM((SEG * L,), f32),
        pltpu.VMEM((SEG * L,), jnp.int32),
        pltpu.VMEM((5 * L,), f32),
        pltpu.VMEM((TOPK * L,), f32),
        pltpu.VMEM((TOPK * L,), f32),
        pltpu.VMEM((TOPK * L,), f32),
        pltpu.VMEM((TOPK * L,), f32),
        pltpu.SemaphoreType.DMA,
    ]
    fn = pl.kernel(_sc_body, out_type=out_type, mesh=mesh,
                   compiler_params=pltpu.CompilerParams(
                       needs_layout_passes=False),
                   scratch_types=scratch)
    return fn(s_sem, s_struct)


def _tc_body(part_ref, sval_ref, sidx_ref, tval_ref, tidx_ref,
             relid_ref, dirid_ref, rel_emb_ref, dir_emb_ref,
             w1_ref, b1_ref, w2_ref, b2_ref, inv_ref, out_ref):
    nf = jnp.float32(N)
    part = part_ref[...]
    s = part[:, 0:L]
    q = part[:, L:2 * L]
    ml = part[:, 2 * L:3 * L]
    s1l = part[:, 3 * L:4 * L]
    s2l = part[:, 4 * L:5 * L]

    mean = jnp.sum(s, axis=1) / nf
    var = jnp.sum(q, axis=1) / nf - mean * mean
    std = jnp.sqrt(jnp.maximum(var, 0.0))
    maxv = jnp.max(ml, axis=1)
    w = jnp.exp(ml - maxv[:, None])
    s1 = jnp.sum(s1l * w, axis=1)
    s2 = jnp.sum(s2l * w, axis=1)
    ent = maxv + jnp.log(s1) - s2 / s1
    gap = maxv - mean

    def select10(vals, idx):
        v = vals
        sels = []
        for _ in range(TOPK):
            mx = jnp.max(v, axis=1, keepdims=True)
            ci = jnp.where(v == mx, idx, jnp.float32(1e9))
            si = jnp.min(ci, axis=1, keepdims=True)
            sels.append(si)
            v = jnp.where(idx == si, -jnp.inf, v)
        return sels

    semsel = select10(sval_ref[...], sidx_ref[...])
    strsel = jnp.concatenate(select10(tval_ref[...], tidx_ref[...]), axis=1)

    match = jnp.zeros((B,), jnp.float32)
    for i in range(TOPK):
        hit = jnp.max((semsel[i] == strsel).astype(jnp.float32), axis=1)
        match = match + hit
    agree = match * inv_ref[0, 0]

    rel_oh = (relid_ref[...] ==
              lax.broadcasted_iota(jnp.int32, (B, rel_emb_ref.shape[0]), 1)
              ).astype(jnp.float32)
    dir_oh = (dirid_ref[...] ==
              lax.broadcasted_iota(jnp.int32, (B, 2), 1)).astype(jnp.float32)
    rel_vec = jnp.dot(rel_oh, rel_emb_ref[...],
                      preferred_element_type=jnp.float32)
    dir_vec = jnp.dot(dir_oh, dir_emb_ref[...],
                      preferred_element_type=jnp.float32)

    x = jnp.concatenate(
        [mean[:, None], std[:, None], maxv[:, None], gap[:, None],
         ent[:, None], agree[:, None], rel_vec, dir_vec], axis=1)
    h = jnp.maximum(jnp.dot(x, w1_ref[...],
                            preferred_element_type=jnp.float32) + b1_ref[...],
                    0.0)
    z = jnp.dot(h, w2_ref[...], preferred_element_type=jnp.float32) + b2_ref[...]
    r = 1.0 / (1.0 + jnp.exp(-z))
    out_ref[...] = jnp.clip(r, 0.05, 0.95)


def kernel(s_sem, s_struct, rel_ids, dir_ids, topm, rel_emb, dir_emb,
           W1, b1, W2, b2):
    part, sval, sidx, tval, tidx = _sc_stage(s_sem, s_struct)

    inv_topm = (1.0 / jnp.asarray(topm, jnp.float32)).reshape(1, 1)
    relid = rel_ids.astype(jnp.int32).reshape(B, 1)
    dirid = dir_ids.astype(jnp.int32).reshape(B, 1)

    out = pl.pallas_call(
        _tc_body,
        out_shape=jax.ShapeDtypeStruct((B, 1), jnp.float32),
    )(part, sval, sidx, tval, tidx, relid, dirid, rel_emb, dir_emb,
      W1, b1.reshape(1, -1), W2, b2.reshape(1, 1), inv_topm)
    return out[:, 0]


# speedup vs baseline: 4.6109x; 1.3976x over previous
import functools

import jax
import jax.numpy as jnp
from jax import lax
from jax.experimental import pallas as pl
from jax.experimental.pallas import tpu as pltpu
from jax.experimental.pallas import tpu_sc as plsc

B = 128
N = 32768
L = 16
NCH = N // L
NC, NS = 2, 16
NW = NC * NS
RPW = B // NW
TOPK = 10
SCH = 16
SEG = NCH // SCH
SPS = 8
NSUP = SEG // SPS
XUNR = 8

OFF_PART = 0
OFF_SVAL = 80
OFF_SIDX = 240
OFF_TVAL = 400
OFF_TIDX = 560
REC = 720


def _merge_chain(va, ia, vb, ib):
    c = (vb > va) | ((vb == va) & (ib < ia))
    return jnp.where(c, vb, va), jnp.where(c, ib, ia)


def _pass1(buf, seg_val, seg_idx, lane_i, with_stats):
    zero = jnp.zeros((L,), jnp.float32)
    ninf = jnp.full((L,), -jnp.inf, jnp.float32)
    zi = jnp.zeros((L,), jnp.int32)

    def body(sg, carry):
        s0, s1, q0, q1, m = carry
        base = sg * (SCH * L)
        sma, smia, smb, smib = ninf, zi, ninf, zi
        for j in range(SCH):
            x = buf[pl.ds(base + j * L, L)]
            gidx = base + (j * L) + lane_i
            if j % 2 == 0:
                c = x > sma
                sma = jnp.where(c, x, sma)
                smia = jnp.where(c, gidx, smia)
                if with_stats:
                    s0 = s0 + x
                    q0 = q0 + x * x
            else:
                c = x > smb
                smb = jnp.where(c, x, smb)
                smib = jnp.where(c, gidx, smib)
                if with_stats:
                    s1 = s1 + x
                    q1 = q1 + x * x
        sm, smi = _merge_chain(sma, smia, smb, smib)
        m = jnp.maximum(m, sm)
        seg_val[pl.ds(sg * L, L)] = sm
        seg_idx[pl.ds(sg * L, L)] = smi
        return (s0, s1, q0, q1, m)

    s0, s1, q0, q1, m = lax.fori_loop(0, SEG, body,
                                      (zero, zero, zero, zero, ninf))
    return s0 + s1, q0 + q1, m


def _build_supseg(seg_val, supseg_val):
    def body(t, _):
        v = seg_val[pl.ds(t * (SPS * L), L)]
        for j in range(1, SPS):
            v = jnp.maximum(v, seg_val[pl.ds(t * (SPS * L) + j * L, L)])
        supseg_val[pl.ds(t * L, L)] = v
        return 0

    lax.fori_loop(0, NSUP, body, 0)


def _softmax_partials(buf, m):
    zero = jnp.zeros((L,), jnp.float32)

    def body(g, carry):
        a0, a1, b0, b1 = carry
        base = g * (XUNR * L)
        for j in range(XUNR):
            x = buf[pl.ds(base + j * L, L)]
            e = jnp.exp(x - m)
            if j % 2 == 0:
                a0 = a0 + e
                b0 = b0 + e * x
            else:
                a1 = a1 + e
                b1 = b1 + e * x
        return (a0, a1, b0, b1)

    a0, a1, b0, b1 = lax.fori_loop(0, NCH // XUNR, body,
                                   (zero, zero, zero, zero))
    return a0 + a1, b0 + b1


def _extract10(buf, seg_val, seg_idx, supseg_val, lane_i, stage,
               val_off, idx_off):
    ninf = jnp.full((L,), -jnp.inf, jnp.float32)
    zi = jnp.zeros((L,), jnp.int32)

    def body(k, _):
        bv, bt = ninf, zi
        for t in range(NSUP):
            v = supseg_val[pl.ds(t * L, L)]
            c = v > bv
            bv = jnp.where(c, v, bv)
            bt = jnp.where(c, zi + t, bt)
        dv, bs = ninf, zi
        for j in range(SPS):
            sj = bt * SPS + j
            g = plsc.load_gather(seg_val, [sj * L + lane_i])
            c = g > dv
            dv = jnp.where(c, g, dv)
            bs = jnp.where(c, sj, bs)
        bi = plsc.load_gather(seg_idx, [bs * L + lane_i])
        plsc.store_scatter(buf, [bi], ninf)
        stage[pl.ds(val_off + k * L, L)] = bv
        stage[pl.ds(idx_off + k * L, L)] = bi.astype(jnp.float32)
        sbase = bs * (SCH * L) + lane_i
        nva, nia, nvb, nib = ninf, zi, ninf, zi
        for j in range(SCH):
            gidx = sbase + j * L
            g = plsc.load_gather(buf, [gidx])
            if j % 2 == 0:
                c = g > nva
                nva = jnp.where(c, g, nva)
                nia = jnp.where(c, gidx, nia)
            else:
                c = g > nvb
                nvb = jnp.where(c, g, nvb)
                nib = jnp.where(c, gidx, nib)
        nv, nvi = _merge_chain(nva, nia, nvb, nib)
        plsc.store_scatter(seg_val, [bs * L + lane_i], nv)
        plsc.store_scatter(seg_idx, [bs * L + lane_i], nvi)
        sv = plsc.load_gather(seg_val, [(bt * SPS) * L + lane_i])
        for j in range(1, SPS):
            g = plsc.load_gather(seg_val, [(bt * SPS + j) * L + lane_i])
            sv = jnp.maximum(sv, g)
        plsc.store_scatter(supseg_val, [bt * L + lane_i], sv)
        return 0

    lax.fori_loop(0, TOPK, body, 0)


def _sc_body(sem_hbm, struct_hbm, out_hbm,
             sem_a, sem_b, struct_v, seg_val, seg_idx, supseg_val,
             stage0, stage1, stage2, stage3, ds_sem, ds_str, ds_out):
    wid = lax.axis_index("s") * NC + lax.axis_index("c")
    lane_i = lax.broadcasted_iota(jnp.int32, (L,), 0)
    r0 = wid * RPW

    sem_bufs = [sem_a, sem_b]
    stage_bufs = [stage0, stage1, stage2, stage3]
    cp_sem = pltpu.async_copy(sem_hbm.at[r0], sem_a, ds_sem)
    out_cps = []
    for rr in range(RPW):
        r = r0 + rr
        cur = sem_bufs[rr % 2]
        stage = stage_bufs[rr]
        cp_struct = pltpu.async_copy(struct_hbm.at[r], struct_v, ds_str)
        cp_sem.wait()
        if rr + 1 < RPW:
            cp_sem = pltpu.async_copy(sem_hbm.at[r + 1],
                                      sem_bufs[(rr + 1) % 2], ds_sem)

        s, q, m = _pass1(cur, seg_val, seg_idx, lane_i, with_stats=True)
        s1, s2 = _softmax_partials(cur, m)
        stage[pl.ds(OFF_PART + 0, L)] = s
        stage[pl.ds(OFF_PART + L, L)] = q
        stage[pl.ds(OFF_PART + 2 * L, L)] = m
        stage[pl.ds(OFF_PART + 3 * L, L)] = s1
        stage[pl.ds(OFF_PART + 4 * L, L)] = s2
        _build_supseg(seg_val, supseg_val)
        _extract10(cur, seg_val, seg_idx, supseg_val, lane_i, stage,
                   OFF_SVAL, OFF_SIDX)

        cp_struct.wait()
        _pass1(struct_v, seg_val, seg_idx, lane_i, with_stats=False)
        _build_supseg(seg_val, supseg_val)
        _extract10(struct_v, seg_val, seg_idx, supseg_val, lane_i, stage,
                   OFF_TVAL, OFF_TIDX)

        out_cps.append(pltpu.async_copy(stage, out_hbm.at[r], ds_out))
    for cp in out_cps:
        cp.wait()


def _sc_stage(s_sem, s_struct):
    mesh = plsc.VectorSubcoreMesh(core_axis_name="c", subcore_axis_name="s",
                                  num_cores=NC, num_subcores=NS)
    f32 = jnp.float32
    scratch = [
        pltpu.VMEM((N,), f32),
        pltpu.VMEM((N,), f32),
        pltpu.VMEM((N,), f32),
        pltpu.VMEM((SEG * L,), f32),
        pltpu.VMEM((SEG * L,), jnp.int32),
        pltpu.VMEM((NSUP * L,), f32),
        pltpu.VMEM((REC,), f32),
        pltpu.VMEM((REC,), f32),
        pltpu.VMEM((REC,), f32),
        pltpu.VMEM((REC,), f32),
        pltpu.SemaphoreType.DMA,
        pltpu.SemaphoreType.DMA,
        pltpu.SemaphoreType.DMA,
    ]
    fn = pl.kernel(_sc_body,
                   out_type=[jax.ShapeDtypeStruct((B, REC), f32)],
                   mesh=mesh,
                   compiler_params=pltpu.CompilerParams(
                       needs_layout_passes=False),
                   scratch_types=scratch)
    return fn(s_sem, s_struct)


def _tc_body(comb_ref, relid_ref, dirid_ref, rel_emb_ref, dir_emb_ref,
             w1_ref, b1_ref, w2_ref, b2_ref, inv_ref, out_ref):
    nf = jnp.float32(N)
    comb = comb_ref[...]
    s = comb[:, 0:L]
    q = comb[:, L:2 * L]
    ml = comb[:, 2 * L:3 * L]
    s1l = comb[:, 3 * L:4 * L]
    s2l = comb[:, 4 * L:5 * L]

    mean = jnp.sum(s, axis=1) / nf
    var = jnp.sum(q, axis=1) / nf - mean * mean
    std = jnp.sqrt(jnp.maximum(var, 0.0))
    maxv = jnp.max(ml, axis=1)
    w = jnp.exp(ml - maxv[:, None])
    s1 = jnp.sum(s1l * w, axis=1)
    s2 = jnp.sum(s2l * w, axis=1)
    ent = maxv + jnp.log(s1) - s2 / s1
    gap = maxv - mean

    def select10(vals, idx):
        v = vals
        sels = []
        for _ in range(TOPK):
            mx = jnp.max(v, axis=1, keepdims=True)
            ci = jnp.where(v == mx, idx, jnp.float32(1e9))
            si = jnp.min(ci, axis=1, keepdims=True)
            sels.append(si)
            v = jnp.where(idx == si, -jnp.inf, v)
        return sels

    semsel = select10(comb[:, OFF_SVAL:OFF_SIDX], comb[:, OFF_SIDX:OFF_TVAL])
    strsel = jnp.concatenate(
        select10(comb[:, OFF_TVAL:OFF_TIDX], comb[:, OFF_TIDX:REC]), axis=1)

    match = jnp.zeros((B,), jnp.float32)
    for i in range(TOPK):
        hit = jnp.max((semsel[i] == strsel).astype(jnp.float32), axis=1)
        match = match + hit
    agree = match * inv_ref[0, 0]

    rel_oh = (relid_ref[...] ==
              lax.broadcasted_iota(jnp.int32, (B, rel_emb_ref.shape[0]), 1)
              ).astype(jnp.float32)
    dir_oh = (dirid_ref[...] ==
              lax.broadcasted_iota(jnp.int32, (B, 2), 1)).astype(jnp.float32)
    rel_vec = jnp.dot(rel_oh, rel_emb_ref[...],
                      preferred_element_type=jnp.float32)
    dir_vec = jnp.dot(dir_oh, dir_emb_ref[...],
                      preferred_element_type=jnp.float32)

    x = jnp.concatenate(
        [mean[:, None], std[:, None], maxv[:, None], gap[:, None],
         ent[:, None], agree[:, None], rel_vec, dir_vec], axis=1)
    h = jnp.maximum(jnp.dot(x, w1_ref[...],
                            preferred_element_type=jnp.float32) + b1_ref[...],
                    0.0)
    z = jnp.dot(h, w2_ref[...], preferred_element_type=jnp.float32) + b2_ref[...]
    r = 1.0 / (1.0 + jnp.exp(-z))
    out_ref[...] = jnp.clip(r, 0.05, 0.95)


def kernel(s_sem, s_struct, rel_ids, dir_ids, topm, rel_emb, dir_emb,
           W1, b1, W2, b2):
    (comb,) = _sc_stage(s_sem, s_struct)

    inv_topm = (1.0 / jnp.asarray(topm, jnp.float32)).reshape(1, 1)
    relid = rel_ids.astype(jnp.int32).reshape(B, 1)
    dirid = dir_ids.astype(jnp.int32).reshape(B, 1)

    out = pl.pallas_call(
        _tc_body,
        out_shape=jax.ShapeDtypeStruct((B, 1), jnp.float32),
    )(comb, relid, dirid, rel_emb, dir_emb,
      W1, b1.reshape(1, -1), W2, b2.reshape(1, 1), inv_topm)
    return out[:, 0]


# speedup vs baseline: 5.5967x; 1.2138x over previous
import functools

import jax
import jax.numpy as jnp
from jax import lax
from jax.experimental import pallas as pl
from jax.experimental.pallas import tpu as pltpu
from jax.experimental.pallas import tpu_sc as plsc

B = 128
N = 32768
L = 16
NCH = N // L
NC, NS = 2, 16
NW = NC * NS
RPW = B // NW
TOPK = 10
SCH = 16
SEG = NCH // SCH
SPS = 8
NSUP = SEG // SPS
RB = 8

OFF_SVAL = 0
OFF_SIDX = 160
OFF_TVAL = 320
OFF_TIDX = 480
REC = 640


def _merge_chain(va, ia, vb, ib):
    c = (vb > va) | ((vb == va) & (ib < ia))
    return jnp.where(c, vb, va), jnp.where(c, ib, ia)


def _pass1(buf, seg_val, seg_idx, lane_i):
    ninf = jnp.full((L,), -jnp.inf, jnp.float32)
    zi = jnp.zeros((L,), jnp.int32)
    jconst = [jnp.full((L,), j * L, jnp.int32) for j in range(SCH)]

    def body(sg, _):
        base = sg * (SCH * L)
        sma, smia, smb, smib = ninf, zi, ninf, zi
        for j in range(SCH):
            x = buf[pl.ds(base + j * L, L)]
            if j % 2 == 0:
                c = x > sma
                sma = jnp.where(c, x, sma)
                smia = jnp.where(c, jconst[j], smia)
            else:
                c = x > smb
                smb = jnp.where(c, x, smb)
                smib = jnp.where(c, jconst[j], smib)
        sm, smi = _merge_chain(sma, smia, smb, smib)
        seg_val[pl.ds(sg * L, L)] = sm
        seg_idx[pl.ds(sg * L, L)] = smi + (base + lane_i)
        return 0

    lax.fori_loop(0, SEG, body, 0)


def _build_supseg(seg_val, supseg_val):
    def body(t, _):
        v = seg_val[pl.ds(t * (SPS * L), L)]
        for j in range(1, SPS):
            v = jnp.maximum(v, seg_val[pl.ds(t * (SPS * L) + j * L, L)])
        supseg_val[pl.ds(t * L, L)] = v
        return 0

    lax.fori_loop(0, NSUP, body, 0)


def _extract10(buf, seg_val, seg_idx, supseg_val, lane_i, stage,
               val_off, idx_off):
    ninf = jnp.full((L,), -jnp.inf, jnp.float32)
    zi = jnp.zeros((L,), jnp.int32)

    def body(k, _):
        bv, bt = ninf, zi
        for t in range(NSUP):
            v = supseg_val[pl.ds(t * L, L)]
            c = v > bv
            bv = jnp.where(c, v, bv)
            bt = jnp.where(c, zi + t, bt)
        dv, bs = ninf, zi
        for j in range(SPS):
            sj = bt * SPS + j
            g = plsc.load_gather(seg_val, [sj * L + lane_i])
            c = g > dv
            dv = jnp.where(c, g, dv)
            bs = jnp.where(c, sj, bs)
        bi = plsc.load_gather(seg_idx, [bs * L + lane_i])
        plsc.store_scatter(buf, [bi], ninf)
        stage[pl.ds(val_off + k * L, L)] = bv
        stage[pl.ds(idx_off + k * L, L)] = bi.astype(jnp.float32)
        sbase = bs * (SCH * L) + lane_i
        nva, nia, nvb, nib = ninf, zi, ninf, zi
        for j in range(SCH):
            gidx = sbase + j * L
            g = plsc.load_gather(buf, [gidx])
            if j % 2 == 0:
                c = g > nva
                nva = jnp.where(c, g, nva)
                nia = jnp.where(c, gidx, nia)
            else:
                c = g > nvb
                nvb = jnp.where(c, g, nvb)
                nib = jnp.where(c, gidx, nib)
        nv, nvi = _merge_chain(nva, nia, nvb, nib)
        plsc.store_scatter(seg_val, [bs * L + lane_i], nv)
        plsc.store_scatter(seg_idx, [bs * L + lane_i], nvi)
        sv = plsc.load_gather(seg_val, [(bt * SPS) * L + lane_i])
        for j in range(1, SPS):
            g = plsc.load_gather(seg_val, [(bt * SPS + j) * L + lane_i])
            sv = jnp.maximum(sv, g)
        plsc.store_scatter(supseg_val, [bt * L + lane_i], sv)
        return 0

    lax.fori_loop(0, TOPK, body, 0)


def _sc_body(sem_hbm, struct_hbm, out_hbm,
             sem_a, sem_b, struct_v, seg_val, seg_idx, supseg_val,
             stage0, stage1, stage2, stage3, ds_sem, ds_str, ds_out):
    wid = lax.axis_index("s") * NC + lax.axis_index("c")
    lane_i = lax.broadcasted_iota(jnp.int32, (L,), 0)
    r0 = wid * RPW

    sem_bufs = [sem_a, sem_b]
    stage_bufs = [stage0, stage1, stage2, stage3]
    cp_sem = pltpu.async_copy(sem_hbm.at[r0], sem_a, ds_sem)
    out_cps = []
    for rr in range(RPW):
        r = r0 + rr
        cur = sem_bufs[rr % 2]
        stage = stage_bufs[rr]
        cp_struct = pltpu.async_copy(struct_hbm.at[r], struct_v, ds_str)
        cp_sem.wait()
        if rr + 1 < RPW:
            cp_sem = pltpu.async_copy(sem_hbm.at[r + 1],
                                      sem_bufs[(rr + 1) % 2], ds_sem)

        _pass1(cur, seg_val, seg_idx, lane_i)
        _build_supseg(seg_val, supseg_val)
        _extract10(cur, seg_val, seg_idx, supseg_val, lane_i, stage,
                   OFF_SVAL, OFF_SIDX)

        cp_struct.wait()
        _pass1(struct_v, seg_val, seg_idx, lane_i)
        _build_supseg(seg_val, supseg_val)
        _extract10(struct_v, seg_val, seg_idx, supseg_val, lane_i, stage,
                   OFF_TVAL, OFF_TIDX)

        out_cps.append(pltpu.async_copy(stage, out_hbm.at[r], ds_out))
    for cp in out_cps:
        cp.wait()


def _sc_stage(s_sem, s_struct):
    mesh = plsc.VectorSubcoreMesh(core_axis_name="c", subcore_axis_name="s",
                                  num_cores=NC, num_subcores=NS)
    f32 = jnp.float32
    scratch = [
        pltpu.VMEM((N,), f32),
        pltpu.VMEM((N,), f32),
        pltpu.VMEM((N,), f32),
        pltpu.VMEM((SEG * L,), f32),
        pltpu.VMEM((SEG * L,), jnp.int32),
        pltpu.VMEM((NSUP * L,), f32),
        pltpu.VMEM((REC,), f32),
        pltpu.VMEM((REC,), f32),
        pltpu.VMEM((REC,), f32),
        pltpu.VMEM((REC,), f32),
        pltpu.SemaphoreType.DMA,
        pltpu.SemaphoreType.DMA,
        pltpu.SemaphoreType.DMA,
    ]
    fn = pl.kernel(_sc_body,
                   out_type=[jax.ShapeDtypeStruct((B, REC), f32)],
                   mesh=mesh,
                   compiler_params=pltpu.CompilerParams(
                       needs_layout_passes=False),
                   scratch_types=scratch)
    return fn(s_sem, s_struct)


def _stats_body(x_ref, out_ref):
    nf = jnp.float32(N)
    x = x_ref[...]
    m = jnp.max(x, axis=1, keepdims=True)
    mean = jnp.sum(x, axis=1, keepdims=True) / nf
    var = jnp.sum(x * x, axis=1, keepdims=True) / nf - mean * mean
    std = jnp.sqrt(jnp.maximum(var, 0.0))
    e = jnp.exp(x - m)
    s1 = jnp.sum(e, axis=1, keepdims=True)
    s2 = jnp.sum(e * x, axis=1, keepdims=True)
    ent = m + jnp.log(s1) - s2 / s1
    gap = m - mean
    z = jnp.zeros_like(mean)
    out_ref[...] = jnp.concatenate(
        [mean, std, m, gap, ent, z, z, z], axis=1)


def _stats_stage(s_sem):
    return pl.pallas_call(
        _stats_body,
        grid=(B // RB,),
        in_specs=[pl.BlockSpec((RB, N), lambda i: (i, 0))],
        out_specs=pl.BlockSpec((RB, 8), lambda i: (i, 0)),
        out_shape=jax.ShapeDtypeStruct((B, 8), jnp.float32),
    )(s_sem)


def _tc_body(stats_ref, comb_ref, relid_ref, dirid_ref, rel_emb_ref,
             dir_emb_ref, w1_ref, b1_ref, w2_ref, b2_ref, inv_ref, out_ref):
    comb = comb_ref[...]
    stats = stats_ref[...]

    def select10(vals, idx):
        v = vals
        sels = []
        for _ in range(TOPK):
            mx = jnp.max(v, axis=1, keepdims=True)
            ci = jnp.where(v == mx, idx, jnp.float32(1e9))
            si = jnp.min(ci, axis=1, keepdims=True)
            sels.append(si)
            v = jnp.where(idx == si, -jnp.inf, v)
        return sels

    semsel = select10(comb[:, OFF_SVAL:OFF_SIDX], comb[:, OFF_SIDX:OFF_TVAL])
    strsel = jnp.concatenate(
        select10(comb[:, OFF_TVAL:OFF_TIDX], comb[:, OFF_TIDX:REC]), axis=1)

    match = jnp.zeros((B,), jnp.float32)
    for i in range(TOPK):
        hit = jnp.max((semsel[i] == strsel).astype(jnp.float32), axis=1)
        match = match + hit
    agree = match * inv_ref[0, 0]

    rel_oh = (relid_ref[...] ==
              lax.broadcasted_iota(jnp.int32, (B, rel_emb_ref.shape[0]), 1)
              ).astype(jnp.float32)
    dir_oh = (dirid_ref[...] ==
              lax.broadcasted_iota(jnp.int32, (B, 2), 1)).astype(jnp.float32)
    rel_vec = jnp.dot(rel_oh, rel_emb_ref[...],
                      preferred_element_type=jnp.float32)
    dir_vec = jnp.dot(dir_oh, dir_emb_ref[...],
                      preferred_element_type=jnp.float32)

    x = jnp.concatenate(
        [stats[:, 0:5], agree[:, None], rel_vec, dir_vec], axis=1)
    h = jnp.maximum(jnp.dot(x, w1_ref[...],
                            preferred_element_type=jnp.float32) + b1_ref[...],
                    0.0)
    z = jnp.dot(h, w2_ref[...], preferred_element_type=jnp.float32) + b2_ref[...]
    r = 1.0 / (1.0 + jnp.exp(-z))
    out_ref[...] = jnp.clip(r, 0.05, 0.95)


def kernel(s_sem, s_struct, rel_ids, dir_ids, topm, rel_emb, dir_emb,
           W1, b1, W2, b2):
    (comb,) = _sc_stage(s_sem, s_struct)
    stats = _stats_stage(s_sem)

    inv_topm = (1.0 / jnp.asarray(topm, jnp.float32)).reshape(1, 1)
    relid = rel_ids.astype(jnp.int32).reshape(B, 1)
    dirid = dir_ids.astype(jnp.int32).reshape(B, 1)

    out = pl.pallas_call(
        _tc_body,
        out_shape=jax.ShapeDtypeStruct((B, 1), jnp.float32),
    )(stats, comb, relid, dirid, rel_emb, dir_emb,
      W1, b1.reshape(1, -1), W2, b2.reshape(1, 1), inv_topm)
    return out[:, 0]
